# baseline (speedup 1.0000x reference)
import functools

import jax
import jax.numpy as jnp
from jax import lax
from jax.experimental import pallas as pl
from jax.experimental.pallas import tpu as pltpu
from jax.experimental.pallas import tpu_sc as plsc

N_CELL = 10000
N_TISSUE = 1000
FDIM = 512
HID = 256
OUT = 256
LOC = 32
CLASSES = 7
NHEAD = 8
R = 10.0
K = 8


def _lin(x, w, b):
    return x @ w.T + b


def _lrelu(x):
    return jax.nn.leaky_relu(x, 0.01)


def _bn(x, g, b):
    m = x.mean(0, keepdims=True)
    v = x.var(0, keepdims=True)
    return (x - m) / jnp.sqrt(v + 1e-5) * g + b


def _graph_norm(x, g, b, a):
    m = x.mean(0, keepdims=True)
    o = x - a * m
    v = (o * o).mean(0, keepdims=True)
    return o / jnp.sqrt(v + 1e-5) * g + b


def _layer_norm(x, g, b):
    m = x.mean(-1, keepdims=True)
    v = x.var(-1, keepdims=True)
    return (x - m) / jnp.sqrt(v + 1e-5) * g + b


def _bdot(a, b):
    return jax.lax.dot_general(
        a.astype(jnp.bfloat16), b.astype(jnp.bfloat16),
        (((1,), (1,)), ((), ())), preferred_element_type=jnp.float32)


def _bdot_acc(a, b, csz=256):
    kdim = a.shape[1]
    if kdim <= csz:
        return _bdot(a, b)
    acc = _bdot(a[:, :csz], b[:, :csz])
    for c in range(csz, kdim, csz):
        acc = acc + _bdot(a[:, c:c + csz], b[:, c:c + csz])
    return acc


def _tower_kern(x_ref, w1_ref, b1_ref, w2_ref, b2_ref, o_ref):
    h = _bdot_acc(x_ref[...], w1_ref[...]) + b1_ref[...]
    h = jax.nn.leaky_relu(h, 0.01)
    o_ref[...] = _bdot_acc(h, w2_ref[...]) + b2_ref[...]


def _attr_tower(x, w1, b1, w2, b2, block=1000):
    n = x.shape[0]
    grid = (n // block,)
    return pl.pallas_call(
        _tower_kern,
        grid=grid,
        in_specs=[
            pl.BlockSpec((block, x.shape[1]), lambda i: (i, 0)),
            pl.BlockSpec(w1.shape, lambda i: (0, 0)),
            pl.BlockSpec(b1.shape, lambda i: (0,)),
            pl.BlockSpec(w2.shape, lambda i: (0, 0)),
            pl.BlockSpec(b2.shape, lambda i: (0,)),
        ],
        out_specs=pl.BlockSpec((block, w2.shape[0]), lambda i: (i, 0)),
        out_shape=jax.ShapeDtypeStruct((n, w2.shape[0]), jnp.float32),
    )(x, w1, b1, w2, b2)


_INT_MAX = 2**31 - 1


def _topk_kern(n, n_tiles, rb, ct, r2, featb_ref, feat_ref, sqb_ref, sq_ref,
               idx_ref, valid_ref):
    i = pl.program_id(0)
    q = featb_ref[...]
    sq_q = sqb_ref[...]

    def body(t, carry):
        run_v, run_i = carry
        g = feat_ref[pl.ds(pl.multiple_of(t * ct, ct), ct), :]
        sq_g = sq_ref[:, pl.ds(pl.multiple_of(t * ct, ct), ct)]
        qg = _bdot(q, g)
        d2 = jnp.maximum((sq_q + sq_g) - 2.0 * qg, 0.0)
        col = t * ct + jax.lax.broadcasted_iota(jnp.int32, (1, ct), 1)
        d2 = jnp.where(col < n, d2, jnp.inf)
        cv = jnp.concatenate([run_v, d2], axis=1)
        ci = jnp.concatenate([run_i, jnp.broadcast_to(col, d2.shape)], axis=1)
        nv, ni = [], []
        for _ in range(K):
            m = jnp.min(cv, axis=1, keepdims=True)
            sel = cv == m
            am = jnp.min(jnp.where(sel, ci, _INT_MAX), axis=1, keepdims=True)
            nv.append(m)
            ni.append(am)
            cv = jnp.where(sel & (ci == am), jnp.inf, cv)
        return jnp.concatenate(nv, axis=1), jnp.concatenate(ni, axis=1)

    run_v = jnp.full((rb, K), jnp.inf, jnp.float32)
    run_i = jnp.zeros((rb, K), jnp.int32)
    run_v, run_i = jax.lax.fori_loop(0, n_tiles, body, (run_v, run_i))
    rows = i * rb + jax.lax.broadcasted_iota(jnp.int32, (rb, 1), 0)
    idx_ref[...] = run_i
    valid_ref[...] = ((run_v <= r2) & (run_i != rows)).astype(jnp.float32)


def _radius_topk(feat, r, rb, ct):
    n, f = feat.shape
    n_tiles = -(-n // ct)
    n_pad = n_tiles * ct
    sq = jnp.sum(feat * feat, axis=1)
    featp = jnp.concatenate([feat, jnp.zeros((n_pad - n, f), feat.dtype)]) if n_pad > n else feat
    sqp = jnp.concatenate([sq, jnp.zeros((n_pad - n,), sq.dtype)]) if n_pad > n else sq
    kern = functools.partial(_topk_kern, n, n_tiles, rb, ct, r * r)
    return pl.pallas_call(
        kern,
        grid=(n // rb,),
        in_specs=[
            pl.BlockSpec((rb, f), lambda i: (i, 0)),
            pl.BlockSpec((n_pad, f), lambda i: (0, 0)),
            pl.BlockSpec((rb, 1), lambda i: (i, 0)),
            pl.BlockSpec((1, n_pad), lambda i: (0, 0)),
        ],
        out_specs=[
            pl.BlockSpec((rb, K), lambda i: (i, 0)),
            pl.BlockSpec((rb, K), lambda i: (i, 0)),
        ],
        out_shape=[
            jax.ShapeDtypeStruct((n, K), jnp.int32),
            jax.ShapeDtypeStruct((n, K), jnp.float32),
        ],
    )(feat, featp, sq[:, None], sqp[None, :])


_NC, _NS = 2, 16
_NW = _NC * _NS


def _sc_gather_sum(table, idx_flat, n_pad, f, rb=16):
    rpw = n_pad // _NW
    nb = rpw // rb
    assert nb % 2 == 0

    @functools.partial(
        pl.kernel,
        mesh=plsc.VectorSubcoreMesh(core_axis_name="c", subcore_axis_name="s"),
        out_type=jax.ShapeDtypeStruct((n_pad, f), jnp.float32),
        scratch_types=[
            pltpu.VMEM((rpw * K,), jnp.int32),
            pltpu.VMEM((rb * K, f), jnp.float32),
            pltpu.VMEM((rb * K, f), jnp.float32),
            pltpu.VMEM((rb, f), jnp.float32),
            pltpu.SemaphoreType.DMA,
            pltpu.SemaphoreType.DMA,
        ],
    )
    def kern(table_hbm, idx_hbm, out_hbm, idx_v, rows0_v, rows1_v, out_v,
             sem0, sem1):
        wid = lax.axis_index("s") * _NC + lax.axis_index("c")
        base = wid * rpw
        pltpu.sync_copy(idx_hbm.at[pl.ds(base * K, rpw * K)], idx_v)

        def gather(bi, rows_v, sem):
            return pltpu.async_copy(
                table_hbm.at[idx_v.at[pl.ds(bi * rb * K, rb * K)]], rows_v, sem)

        def compute(bi, rows_v):
            for r in range(rb):
                for c in range(f // 16):
                    sl = pl.ds(c * 16, 16)
                    v = rows_v[r * K, sl]
                    for j in range(1, K):
                        v = v + rows_v[r * K + j, sl]
                    out_v[r, sl] = v
            pltpu.sync_copy(out_v, out_hbm.at[pl.ds(base + bi * rb, rb)])

        gather(0, rows0_v, sem0)
        gather(1, rows1_v, sem1)

        def pair(t, carry):
            pltpu.make_async_copy(
                table_hbm.at[idx_v.at[pl.ds(0, rb * K)]], rows0_v, sem0).wait()
            compute(2 * t, rows0_v)

            @pl.when(t < nb // 2 - 1)
            def _():
                gather(2 * t + 2, rows0_v, sem0)

            pltpu.make_async_copy(
                table_hbm.at[idx_v.at[pl.ds(0, rb * K)]], rows1_v, sem1).wait()
            compute(2 * t + 1, rows1_v)

            @pl.when(t < nb // 2 - 1)
            def _():
                gather(2 * t + 3, rows1_v, sem1)

            return carry

        lax.fori_loop(0, nb // 2, pair, 0)

    return kern(table, idx_flat)


def _sage_sc(x, idx, valid, wl, wr, b, n, n_pad):
    z = x @ wl.T
    zz = jnp.concatenate([z, jnp.zeros((1, z.shape[1]), z.dtype)], 0)
    idxr = jnp.where(valid > 0.0, idx, n).astype(jnp.int32)
    idxp = jnp.concatenate(
        [idxr, jnp.full((n_pad - n, K), n, jnp.int32)], 0).reshape(-1)
    s = _sc_gather_sum(zz, idxp, n_pad, z.shape[1])[:n]
    c = valid.sum(1)
    mean = s / jnp.maximum(c, 1.0)[:, None]
    return mean + x @ wr.T + b


def _sage(x, idx, valid, wl, wr, b, n):
    z = x @ wl.T
    msgs = z[idx.reshape(-1)].reshape(n, K, z.shape[1]) * valid[:, :, None]
    s = msgs.sum(1)
    c = valid.sum(1)
    mean = s / jnp.maximum(c, 1.0)[:, None]
    return mean + x @ wr.T + b


_DH = OUT // NHEAD
_SCALE = float(_DH) ** 0.5

_ENC_PNAMES = []
for _l in range(2):
    _ENC_PNAMES += ['t%d_%s' % (_l, s) for s in
                    ('in_w', 'in_b', 'out_w', 'out_b', 'ln1_g', 'ln1_b',
                     'ff1_w', 'ff1_b', 'ff2_w', 'ff2_b', 'ln2_g', 'ln2_b')]
_ENC_PNAMES += ['att1_w', 'att1_b', 'att2_w', 'att2_b']


def _ln_rows(x, g, b):
    m = jnp.mean(x, axis=1, keepdims=True)
    d = x - m
    v = jnp.mean(d * d, axis=1, keepdims=True)
    return d * jax.lax.rsqrt(v + 1e-5) * g + b


def _enc_kern(cls_ref, pos_ref, xc_ref, xt_ref, *refs):
    pr = {n: r[...] for n, r in zip(_ENC_PNAMES, refs[:len(_ENC_PNAMES)])}
    att_ref, pool_ref = refs[len(_ENC_PNAMES):]
    rb = xc_ref.shape[0]
    lane = jax.lax.broadcasted_iota(jnp.int32, (OUT, NHEAD), 0)
    head = jax.lax.broadcasted_iota(jnp.int32, (OUT, NHEAD), 1)
    G = (lane // _DH == head).astype(jnp.float32)

    t = [jnp.broadcast_to(cls_ref[...] + pos_ref[0:1, :], (rb, OUT)),
         xc_ref[...] + pos_ref[1:2, :],
         xt_ref[...] + pos_ref[2:3, :]]

    for l in range(2):
        in_w = pr['t%d_in_w' % l]
        wq, wk, wv = in_w[:OUT], in_w[OUT:2 * OUT], in_w[2 * OUT:]
        in_b = pr['t%d_in_b' % l]
        bq, bk, bv = in_b[:OUT], in_b[OUT:2 * OUT], in_b[2 * OUT:]
        q = [_bdot(ti, wq) + bq for ti in t]
        k = [_bdot(ti, wk) + bk for ti in t]
        v = [_bdot(ti, wv) + bv for ti in t]
        logit = [[jax.lax.dot_general(
            q[i] * k[j], G, (((1,), (0,)), ((), ())),
            preferred_element_type=jnp.float32) / _SCALE
            for j in range(3)] for i in range(3)]
        a = []
        for i in range(3):
            m = jnp.maximum(jnp.maximum(logit[i][0], logit[i][1]), logit[i][2])
            e = [jnp.exp(logit[i][j] - m) for j in range(3)]
            tot = e[0] + e[1] + e[2]
            o = None
            for j in range(3):
                s_full = jax.lax.dot_general(
                    e[j] / tot, G, (((1,), (1,)), ((), ())),
                    preferred_element_type=jnp.float32)
                term = s_full * v[j]
                o = term if o is None else o + term
            a.append(_bdot(o, pr['t%d_out_w' % l]) + pr['t%d_out_b' % l])
        x = [_ln_rows(t[i] + a[i], pr['t%d_ln1_g' % l], pr['t%d_ln1_b' % l])
             for i in range(3)]
        t = []
        for i in range(3):
            f = jnp.maximum(_bdot(x[i], pr['t%d_ff1_w' % l]) + pr['t%d_ff1_b' % l], 0.0)
            f = _bdot(f, pr['t%d_ff2_w' % l]) + pr['t%d_ff2_b' % l]
            t.append(_ln_rows(x[i] + f, pr['t%d_ln2_g' % l], pr['t%d_ln2_b' % l]))

    feat = t[0]
    h = jax.nn.leaky_relu(_bdot(feat, pr['att1_w']) + pr['att1_b'], 0.01)
    att_ref[...] = jnp.sum(h * pr['att2_w'], axis=1, keepdims=True) + pr['att2_b']
    pool_ref[...] = jnp.sum(feat, axis=0, keepdims=True)[None]


def _encoder_head(xc, xtg, p, rb=1000):
    n = xc.shape[0]
    grid = (n // rb,)
    enc_params = [p[name] for name in _ENC_PNAMES]
    cls2 = p['cls'].reshape(1, OUT)
    pos2 = p['pos'].reshape(3, OUT)
    full = lambda a: pl.BlockSpec(a.shape, lambda i: (0,) * a.ndim)
    att_raw, pools = pl.pallas_call(
        _enc_kern,
        grid=grid,
        in_specs=[
            full(cls2),
            full(pos2),
            pl.BlockSpec((rb, OUT), lambda i: (i, 0)),
            pl.BlockSpec((rb, OUT), lambda i: (i, 0)),
        ] + [full(a) for a in enc_params],
        out_specs=[
            pl.BlockSpec((rb, 1), lambda i: (i, 0)),
            pl.BlockSpec((1, 1, OUT), lambda i: (i, 0, 0)),
        ],
        out_shape=[
            jax.ShapeDtypeStruct((n, 1), jnp.float32),
            jax.ShapeDtypeStruct((grid[0], 1, OUT), jnp.float32),
        ],
    )(cls2, pos2, xc, xtg, *enc_params)
    return att_raw, pools.reshape(grid[0], OUT)


def kernel(x_cell, centroids_cell, x_tissue_3, centroids_tissue_3, assignment_matrix_3, params):
    p = params
    batch_idx = jnp.argmax(assignment_matrix_3, axis=1)

    ca = _attr_tower(x_cell, p['ca1_w'], p['ca1_b'], p['ca2_w'], p['ca2_b'])
    cl = _bn(centroids_cell, p['cl_bn_g'], p['cl_bn_b'])
    cl = _lin(_lrelu(_lin(cl, p['cl1_w'], p['cl1_b'])), p['cl2_w'], p['cl2_b'])
    cell_feat = jnp.concatenate([ca, cl], axis=1)
    ta = _attr_tower(x_tissue_3, p['ta1_w'], p['ta1_b'], p['ta2_w'], p['ta2_b'])
    tl = _bn(centroids_tissue_3, p['tl_bn_g'], p['tl_bn_b'])
    tl = _lin(_lrelu(_lin(tl, p['tl1_w'], p['tl1_b'])), p['tl2_w'], p['tl2_b'])
    tis_feat = jnp.concatenate([ta, tl], axis=1)

    c_idx, c_valid = _radius_topk(cell_feat, R, rb=1000, ct=512)
    t_idx, t_valid = _radius_topk(tis_feat, R, rb=1000, ct=512)

    xc = x_cell
    for j in (1, 2, 3):
        xc = _sage_sc(xc, c_idx, c_valid, p['c%d_wl' % j], p['c%d_wr' % j], p['c%d_b' % j],
                      N_CELL, 10240)
        xc = _lrelu(_graph_norm(xc, p['gn_g'], p['gn_b'], p['gn_a']))
    xt = x_tissue_3
    for j in (4, 5, 6):
        xt = _sage_sc(xt, t_idx, t_valid, p['c%d_wl' % j], p['c%d_wr' % j], p['c%d_b' % j],
                      N_TISSUE, 1024)
        xt = _lrelu(_graph_norm(xt, p['gn_g'], p['gn_b'], p['gn_a']))

    xtg = xt[batch_idx]
    att_raw, pools = _encoder_head(xc, xtg, p)
    att = jax.nn.softmax(att_raw, axis=0)
    pooled = jnp.sum(pools, axis=0, keepdims=True) * (1.0 / N_CELL)
    h = _lrelu(_lin(pooled, p['lin1_w'], p['lin1_b']))
    h = _layer_norm(h, p['ln2_g'], p['ln2_b'])
    logits = _lin(h, p['lin2_w'], p['lin2_b'])
    c_src = c_idx.reshape(-1)
    c_dst = jnp.broadcast_to(
        jnp.arange(N_CELL, dtype=c_idx.dtype)[:, None], (N_CELL, K)).reshape(-1)
    edge_index_cell = jnp.stack([c_src, c_dst])
    return logits, edge_index_cell, att


# speedup vs baseline: 1.0075x; 1.0075x over previous
import functools

import jax
import jax.numpy as jnp
from jax import lax
from jax.experimental import pallas as pl
from jax.experimental.pallas import tpu as pltpu
from jax.experimental.pallas import tpu_sc as plsc

N_CELL = 10000
N_TISSUE = 1000
FDIM = 512
HID = 256
OUT = 256
LOC = 32
CLASSES = 7
NHEAD = 8
R = 10.0
K = 8


def _lin(x, w, b):
    return x @ w.T + b


def _lrelu(x):
    return jax.nn.leaky_relu(x, 0.01)


def _bn(x, g, b):
    m = x.mean(0, keepdims=True)
    v = x.var(0, keepdims=True)
    return (x - m) / jnp.sqrt(v + 1e-5) * g + b


def _graph_norm(x, g, b, a):
    m = x.mean(0, keepdims=True)
    o = x - a * m
    v = (o * o).mean(0, keepdims=True)
    return o / jnp.sqrt(v + 1e-5) * g + b


def _layer_norm(x, g, b):
    m = x.mean(-1, keepdims=True)
    v = x.var(-1, keepdims=True)
    return (x - m) / jnp.sqrt(v + 1e-5) * g + b


def _bdot(a, b):
    return jax.lax.dot_general(
        a.astype(jnp.bfloat16), b.astype(jnp.bfloat16),
        (((1,), (1,)), ((), ())), preferred_element_type=jnp.float32)


def _bdot_acc(a, b, csz=256):
    kdim = a.shape[1]
    if kdim <= csz:
        return _bdot(a, b)
    acc = _bdot(a[:, :csz], b[:, :csz])
    for c in range(csz, kdim, csz):
        acc = acc + _bdot(a[:, c:c + csz], b[:, c:c + csz])
    return acc


def _tower_kern(x_ref, w1_ref, b1_ref, w2_ref, b2_ref, o_ref):
    h = _bdot_acc(x_ref[...], w1_ref[...]) + b1_ref[...]
    h = jax.nn.leaky_relu(h, 0.01)
    o_ref[...] = _bdot_acc(h, w2_ref[...]) + b2_ref[...]


def _attr_tower(x, w1, b1, w2, b2, block=1000):
    n = x.shape[0]
    grid = (n // block,)
    return pl.pallas_call(
        _tower_kern,
        grid=grid,
        in_specs=[
            pl.BlockSpec((block, x.shape[1]), lambda i: (i, 0)),
            pl.BlockSpec(w1.shape, lambda i: (0, 0)),
            pl.BlockSpec(b1.shape, lambda i: (0,)),
            pl.BlockSpec(w2.shape, lambda i: (0, 0)),
            pl.BlockSpec(b2.shape, lambda i: (0,)),
        ],
        out_specs=pl.BlockSpec((block, w2.shape[0]), lambda i: (i, 0)),
        out_shape=jax.ShapeDtypeStruct((n, w2.shape[0]), jnp.float32),
    )(x, w1, b1, w2, b2)


_INT_MAX = 2**31 - 1


def _topk_kern(n, n_tiles, rb, ct, r2, featb_ref, feat_ref, sqb_ref, sq_ref,
               idx_ref, valid_ref):
    i = pl.program_id(0)
    q = featb_ref[...]
    sq_q = sqb_ref[...]

    def body(t, carry):
        run_v, run_i = carry
        g = feat_ref[pl.ds(pl.multiple_of(t * ct, ct), ct), :]
        sq_g = sq_ref[:, pl.ds(pl.multiple_of(t * ct, ct), ct)]
        qg = _bdot(q, g)
        d2 = jnp.maximum((sq_q + sq_g) - 2.0 * qg, 0.0)
        col = t * ct + jax.lax.broadcasted_iota(jnp.int32, (1, ct), 1)
        d2 = jnp.where(col < n, d2, jnp.inf)
        cv = jnp.concatenate([run_v, d2], axis=1)
        ci = jnp.concatenate([run_i, jnp.broadcast_to(col, d2.shape)], axis=1)
        nv, ni = [], []
        for _ in range(K):
            m = jnp.min(cv, axis=1, keepdims=True)
            sel = cv == m
            am = jnp.min(jnp.where(sel, ci, _INT_MAX), axis=1, keepdims=True)
            nv.append(m)
            ni.append(am)
            cv = jnp.where(sel & (ci == am), jnp.inf, cv)
        return jnp.concatenate(nv, axis=1), jnp.concatenate(ni, axis=1)

    run_v = jnp.full((rb, K), jnp.inf, jnp.float32)
    run_i = jnp.zeros((rb, K), jnp.int32)
    run_v, run_i = jax.lax.fori_loop(0, n_tiles, body, (run_v, run_i))
    rows = i * rb + jax.lax.broadcasted_iota(jnp.int32, (rb, 1), 0)
    idx_ref[...] = run_i
    valid_ref[...] = ((run_v <= r2) & (run_i != rows)).astype(jnp.float32)


def _radius_topk(feat, r, rb, ct):
    n, f = feat.shape
    n_tiles = -(-n // ct)
    n_pad = n_tiles * ct
    sq = jnp.sum(feat * feat, axis=1)
    featp = jnp.concatenate([feat, jnp.zeros((n_pad - n, f), feat.dtype)]) if n_pad > n else feat
    sqp = jnp.concatenate([sq, jnp.zeros((n_pad - n,), sq.dtype)]) if n_pad > n else sq
    kern = functools.partial(_topk_kern, n, n_tiles, rb, ct, r * r)
    return pl.pallas_call(
        kern,
        grid=(n // rb,),
        in_specs=[
            pl.BlockSpec((rb, f), lambda i: (i, 0)),
            pl.BlockSpec((n_pad, f), lambda i: (0, 0)),
            pl.BlockSpec((rb, 1), lambda i: (i, 0)),
            pl.BlockSpec((1, n_pad), lambda i: (0, 0)),
        ],
        out_specs=[
            pl.BlockSpec((rb, K), lambda i: (i, 0)),
            pl.BlockSpec((rb, K), lambda i: (i, 0)),
        ],
        out_shape=[
            jax.ShapeDtypeStruct((n, K), jnp.int32),
            jax.ShapeDtypeStruct((n, K), jnp.float32),
        ],
    )(feat, featp, sq[:, None], sqp[None, :])


_NC, _NS = 2, 16
_NW = _NC * _NS


def _sc_gather_sum(table, idx_flat, n_pad, f, rb=16):
    rpw = n_pad // _NW
    nb = rpw // rb
    assert nb % 2 == 0

    @functools.partial(
        pl.kernel,
        mesh=plsc.VectorSubcoreMesh(core_axis_name="c", subcore_axis_name="s"),
        out_type=jax.ShapeDtypeStruct((n_pad, f), jnp.float32),
        scratch_types=[
            pltpu.VMEM((rpw * K,), jnp.int32),
            pltpu.VMEM((rb * K, f), jnp.float32),
            pltpu.VMEM((rb * K, f), jnp.float32),
            pltpu.VMEM((rb, f), jnp.float32),
            pltpu.SemaphoreType.DMA,
            pltpu.SemaphoreType.DMA,
        ],
    )
    def kern(table_hbm, idx_hbm, out_hbm, idx_v, rows0_v, rows1_v, out_v,
             sem0, sem1):
        wid = lax.axis_index("s") * _NC + lax.axis_index("c")
        base = wid * rpw
        pltpu.sync_copy(idx_hbm.at[pl.ds(base * K, rpw * K)], idx_v)

        def gather(bi, rows_v, sem):
            return pltpu.async_copy(
                table_hbm.at[idx_v.at[pl.ds(bi * rb * K, rb * K)]], rows_v, sem)

        def compute(bi, rows_v):
            for r in range(rb):
                for c in range(f // 16):
                    sl = pl.ds(c * 16, 16)
                    v = rows_v[r * K, sl]
                    for j in range(1, K):
                        v = v + rows_v[r * K + j, sl]
                    out_v[r, sl] = v
            pltpu.sync_copy(out_v, out_hbm.at[pl.ds(base + bi * rb, rb)])

        gather(0, rows0_v, sem0)
        gather(1, rows1_v, sem1)

        def pair(t, carry):
            pltpu.make_async_copy(
                table_hbm.at[idx_v.at[pl.ds(0, rb * K)]], rows0_v, sem0).wait()
            compute(2 * t, rows0_v)

            @pl.when(t < nb // 2 - 1)
            def _():
                gather(2 * t + 2, rows0_v, sem0)

            pltpu.make_async_copy(
                table_hbm.at[idx_v.at[pl.ds(0, rb * K)]], rows1_v, sem1).wait()
            compute(2 * t + 1, rows1_v)

            @pl.when(t < nb // 2 - 1)
            def _():
                gather(2 * t + 3, rows1_v, sem1)

            return carry

        lax.fori_loop(0, nb // 2, pair, 0)

    return kern(table, idx_flat)


def _sage_sc_pair(xc, xt, c_idx, c_valid, t_idx, t_valid, jc, jt, p):
    npc, npt = 10240, 1024
    off = N_CELL + 1
    zc = xc @ p['c%d_wl' % jc].T
    zt = xt @ p['c%d_wl' % jt].T
    f = zc.shape[1]
    table = jnp.concatenate(
        [zc, jnp.zeros((1, f), zc.dtype), zt, jnp.zeros((1, f), zt.dtype)], 0)
    ci = jnp.where(c_valid > 0.0, c_idx, N_CELL).astype(jnp.int32)
    ti = off + jnp.where(t_valid > 0.0, t_idx, N_TISSUE).astype(jnp.int32)
    idxp = jnp.concatenate([
        ci.reshape(-1),
        jnp.full(((npc - N_CELL) * K,), N_CELL, jnp.int32),
        ti.reshape(-1),
        jnp.full(((npt - N_TISSUE) * K,), N_CELL, jnp.int32),
    ])
    s = _sc_gather_sum(table, idxp, npc + npt, f)
    cc = c_valid.sum(1)
    ct = t_valid.sum(1)
    mc = s[:N_CELL] / jnp.maximum(cc, 1.0)[:, None]
    mt = s[npc:npc + N_TISSUE] / jnp.maximum(ct, 1.0)[:, None]
    yc = mc + xc @ p['c%d_wr' % jc].T + p['c%d_b' % jc]
    yt = mt + xt @ p['c%d_wr' % jt].T + p['c%d_b' % jt]
    return yc, yt


def _sage(x, idx, valid, wl, wr, b, n):
    z = x @ wl.T
    msgs = z[idx.reshape(-1)].reshape(n, K, z.shape[1]) * valid[:, :, None]
    s = msgs.sum(1)
    c = valid.sum(1)
    mean = s / jnp.maximum(c, 1.0)[:, None]
    return mean + x @ wr.T + b


_DH = OUT // NHEAD
_SCALE = float(_DH) ** 0.5

_ENC_PNAMES = []
for _l in range(2):
    _ENC_PNAMES += ['t%d_%s' % (_l, s) for s in
                    ('in_w', 'in_b', 'out_w', 'out_b', 'ln1_g', 'ln1_b',
                     'ff1_w', 'ff1_b', 'ff2_w', 'ff2_b', 'ln2_g', 'ln2_b')]
_ENC_PNAMES += ['att1_w', 'att1_b', 'att2_w', 'att2_b']


def _ln_rows(x, g, b):
    m = jnp.mean(x, axis=1, keepdims=True)
    d = x - m
    v = jnp.mean(d * d, axis=1, keepdims=True)
    return d * jax.lax.rsqrt(v + 1e-5) * g + b


def _enc_kern(cls_ref, pos_ref, xc_ref, xt_ref, *refs):
    pr = {n: r[...] for n, r in zip(_ENC_PNAMES, refs[:len(_ENC_PNAMES)])}
    att_ref, pool_ref = refs[len(_ENC_PNAMES):]
    rb = xc_ref.shape[0]
    lane = jax.lax.broadcasted_iota(jnp.int32, (OUT, NHEAD), 0)
    head = jax.lax.broadcasted_iota(jnp.int32, (OUT, NHEAD), 1)
    G = (lane // _DH == head).astype(jnp.float32)

    t = [jnp.broadcast_to(cls_ref[...] + pos_ref[0:1, :], (rb, OUT)),
         xc_ref[...] + pos_ref[1:2, :],
         xt_ref[...] + pos_ref[2:3, :]]

    for l in range(2):
        in_w = pr['t%d_in_w' % l]
        wq, wk, wv = in_w[:OUT], in_w[OUT:2 * OUT], in_w[2 * OUT:]
        in_b = pr['t%d_in_b' % l]
        bq, bk, bv = in_b[:OUT], in_b[OUT:2 * OUT], in_b[2 * OUT:]
        q = [_bdot(ti, wq) + bq for ti in t]
        k = [_bdot(ti, wk) + bk for ti in t]
        v = [_bdot(ti, wv) + bv for ti in t]
        logit = [[jax.lax.dot_general(
            q[i] * k[j], G, (((1,), (0,)), ((), ())),
            preferred_element_type=jnp.float32) / _SCALE
            for j in range(3)] for i in range(3)]
        a = []
        for i in range(3):
            m = jnp.maximum(jnp.maximum(logit[i][0], logit[i][1]), logit[i][2])
            e = [jnp.exp(logit[i][j] - m) for j in range(3)]
            tot = e[0] + e[1] + e[2]
            o = None
            for j in range(3):
                s_full = jax.lax.dot_general(
                    e[j] / tot, G, (((1,), (1,)), ((), ())),
                    preferred_element_type=jnp.float32)
                term = s_full * v[j]
                o = term if o is None else o + term
            a.append(_bdot(o, pr['t%d_out_w' % l]) + pr['t%d_out_b' % l])
        x = [_ln_rows(t[i] + a[i], pr['t%d_ln1_g' % l], pr['t%d_ln1_b' % l])
             for i in range(3)]
        t = []
        for i in range(3):
            f = jnp.maximum(_bdot(x[i], pr['t%d_ff1_w' % l]) + pr['t%d_ff1_b' % l], 0.0)
            f = _bdot(f, pr['t%d_ff2_w' % l]) + pr['t%d_ff2_b' % l]
            t.append(_ln_rows(x[i] + f, pr['t%d_ln2_g' % l], pr['t%d_ln2_b' % l]))

    feat = t[0]
    h = jax.nn.leaky_relu(_bdot(feat, pr['att1_w']) + pr['att1_b'], 0.01)
    att_ref[...] = jnp.sum(h * pr['att2_w'], axis=1, keepdims=True) + pr['att2_b']
    pool_ref[...] = jnp.sum(feat, axis=0, keepdims=True)[None]


def _encoder_head(xc, xtg, p, rb=1000):
    n = xc.shape[0]
    grid = (n // rb,)
    enc_params = [p[name] for name in _ENC_PNAMES]
    cls2 = p['cls'].reshape(1, OUT)
    pos2 = p['pos'].reshape(3, OUT)
    full = lambda a: pl.BlockSpec(a.shape, lambda i: (0,) * a.ndim)
    att_raw, pools = pl.pallas_call(
        _enc_kern,
        grid=grid,
        in_specs=[
            full(cls2),
            full(pos2),
            pl.BlockSpec((rb, OUT), lambda i: (i, 0)),
            pl.BlockSpec((rb, OUT), lambda i: (i, 0)),
        ] + [full(a) for a in enc_params],
        out_specs=[
            pl.BlockSpec((rb, 1), lambda i: (i, 0)),
            pl.BlockSpec((1, 1, OUT), lambda i: (i, 0, 0)),
        ],
        out_shape=[
            jax.ShapeDtypeStruct((n, 1), jnp.float32),
            jax.ShapeDtypeStruct((grid[0], 1, OUT), jnp.float32),
        ],
    )(cls2, pos2, xc, xtg, *enc_params)
    return att_raw, pools.reshape(grid[0], OUT)


def kernel(x_cell, centroids_cell, x_tissue_3, centroids_tissue_3, assignment_matrix_3, params):
    p = params
    batch_idx = jnp.argmax(assignment_matrix_3, axis=1)

    ca = _attr_tower(x_cell, p['ca1_w'], p['ca1_b'], p['ca2_w'], p['ca2_b'])
    cl = _bn(centroids_cell, p['cl_bn_g'], p['cl_bn_b'])
    cl = _lin(_lrelu(_lin(cl, p['cl1_w'], p['cl1_b'])), p['cl2_w'], p['cl2_b'])
    cell_feat = jnp.concatenate([ca, cl], axis=1)
    ta = _attr_tower(x_tissue_3, p['ta1_w'], p['ta1_b'], p['ta2_w'], p['ta2_b'])
    tl = _bn(centroids_tissue_3, p['tl_bn_g'], p['tl_bn_b'])
    tl = _lin(_lrelu(_lin(tl, p['tl1_w'], p['tl1_b'])), p['tl2_w'], p['tl2_b'])
    tis_feat = jnp.concatenate([ta, tl], axis=1)

    c_idx, c_valid = _radius_topk(cell_feat, R, rb=1000, ct=512)
    t_idx, t_valid = _radius_topk(tis_feat, R, rb=1000, ct=512)

    xc, xt = x_cell, x_tissue_3
    for jc, jt in ((1, 4), (2, 5), (3, 6)):
        xc, xt = _sage_sc_pair(xc, xt, c_idx, c_valid, t_idx, t_valid, jc, jt, p)
        xc = _lrelu(_graph_norm(xc, p['gn_g'], p['gn_b'], p['gn_a']))
        xt = _lrelu(_graph_norm(xt, p['gn_g'], p['gn_b'], p['gn_a']))

    xtg = xt[batch_idx]
    att_raw, pools = _encoder_head(xc, xtg, p)
    att = jax.nn.softmax(att_raw, axis=0)
    pooled = jnp.sum(pools, axis=0, keepdims=True) * (1.0 / N_CELL)
    h = _lrelu(_lin(pooled, p['lin1_w'], p['lin1_b']))
    h = _layer_norm(h, p['ln2_g'], p['ln2_b'])
    logits = _lin(h, p['lin2_w'], p['lin2_b'])
    c_src = c_idx.reshape(-1)
    c_dst = jnp.broadcast_to(
        jnp.arange(N_CELL, dtype=c_idx.dtype)[:, None], (N_CELL, K)).reshape(-1)
    edge_index_cell = jnp.stack([c_src, c_dst])
    return logits, edge_index_cell, att


# speedup vs baseline: 1.0078x; 1.0002x over previous
import functools

import jax
import jax.numpy as jnp
from jax import lax
from jax.experimental import pallas as pl
from jax.experimental.pallas import tpu as pltpu
from jax.experimental.pallas import tpu_sc as plsc

N_CELL = 10000
N_TISSUE = 1000
FDIM = 512
HID = 256
OUT = 256
LOC = 32
CLASSES = 7
NHEAD = 8
R = 10.0
K = 8


def _lin(x, w, b):
    return x @ w.T + b


def _lrelu(x):
    return jax.nn.leaky_relu(x, 0.01)


def _bn(x, g, b):
    m = x.mean(0, keepdims=True)
    v = x.var(0, keepdims=True)
    return (x - m) / jnp.sqrt(v + 1e-5) * g + b


def _graph_norm(x, g, b, a):
    m = x.mean(0, keepdims=True)
    o = x - a * m
    v = (o * o).mean(0, keepdims=True)
    return o / jnp.sqrt(v + 1e-5) * g + b


def _layer_norm(x, g, b):
    m = x.mean(-1, keepdims=True)
    v = x.var(-1, keepdims=True)
    return (x - m) / jnp.sqrt(v + 1e-5) * g + b


def _bdot(a, b):
    return jax.lax.dot_general(
        a.astype(jnp.bfloat16), b.astype(jnp.bfloat16),
        (((1,), (1,)), ((), ())), preferred_element_type=jnp.float32)


def _bdot_acc(a, b, csz=256):
    kdim = a.shape[1]
    if kdim <= csz:
        return _bdot(a, b)
    acc = _bdot(a[:, :csz], b[:, :csz])
    for c in range(csz, kdim, csz):
        acc = acc + _bdot(a[:, c:c + csz], b[:, c:c + csz])
    return acc


def _tower_kern(x_ref, w1_ref, b1_ref, w2_ref, b2_ref, o_ref):
    h = _bdot_acc(x_ref[...], w1_ref[...]) + b1_ref[...]
    h = jax.nn.leaky_relu(h, 0.01)
    o_ref[...] = _bdot_acc(h, w2_ref[...]) + b2_ref[...]


def _attr_tower(x, w1, b1, w2, b2, block=1000):
    n = x.shape[0]
    grid = (n // block,)
    return pl.pallas_call(
        _tower_kern,
        grid=grid,
        in_specs=[
            pl.BlockSpec((block, x.shape[1]), lambda i: (i, 0)),
            pl.BlockSpec(w1.shape, lambda i: (0, 0)),
            pl.BlockSpec(b1.shape, lambda i: (0,)),
            pl.BlockSpec(w2.shape, lambda i: (0, 0)),
            pl.BlockSpec(b2.shape, lambda i: (0,)),
        ],
        out_specs=pl.BlockSpec((block, w2.shape[0]), lambda i: (i, 0)),
        out_shape=jax.ShapeDtypeStruct((n, w2.shape[0]), jnp.float32),
    )(x, w1, b1, w2, b2)


_INT_MAX = 2**31 - 1


def _topk_kern(n, n_tiles, rb, ct, r2, featb_ref, feat_ref, sqb_ref, sq_ref,
               idx_ref, valid_ref):
    i = pl.program_id(0)
    q = featb_ref[...]
    sq_q = sqb_ref[...]

    def body(t, carry):
        run_v, run_i = carry
        g = feat_ref[pl.ds(pl.multiple_of(t * ct, ct), ct), :]
        sq_g = sq_ref[:, pl.ds(pl.multiple_of(t * ct, ct), ct)]
        qg = _bdot(q, g)
        d2 = jnp.maximum((sq_q + sq_g) - 2.0 * qg, 0.0)
        col = t * ct + jax.lax.broadcasted_iota(jnp.int32, (1, ct), 1)
        d2 = jnp.where(col < n, d2, jnp.inf)
        cv = jnp.concatenate([run_v, d2], axis=1)
        ci = jnp.concatenate([run_i, jnp.broadcast_to(col, d2.shape)], axis=1)
        nv, ni = [], []
        for _ in range(K):
            m = jnp.min(cv, axis=1, keepdims=True)
            sel = cv == m
            am = jnp.min(jnp.where(sel, ci, _INT_MAX), axis=1, keepdims=True)
            nv.append(m)
            ni.append(am)
            cv = jnp.where(sel & (ci == am), jnp.inf, cv)
        return jnp.concatenate(nv, axis=1), jnp.concatenate(ni, axis=1)

    run_v = jnp.full((rb, K), jnp.inf, jnp.float32)
    run_i = jnp.zeros((rb, K), jnp.int32)
    run_v, run_i = jax.lax.fori_loop(0, n_tiles, body, (run_v, run_i))
    rows = i * rb + jax.lax.broadcasted_iota(jnp.int32, (rb, 1), 0)
    idx_ref[...] = run_i
    valid_ref[...] = ((run_v <= r2) & (run_i != rows)).astype(jnp.float32)


def _radius_topk(feat, r, rb, ct):
    n, f = feat.shape
    n_tiles = -(-n // ct)
    n_pad = n_tiles * ct
    sq = jnp.sum(feat * feat, axis=1)
    featp = jnp.concatenate([feat, jnp.zeros((n_pad - n, f), feat.dtype)]) if n_pad > n else feat
    sqp = jnp.concatenate([sq, jnp.zeros((n_pad - n,), sq.dtype)]) if n_pad > n else sq
    kern = functools.partial(_topk_kern, n, n_tiles, rb, ct, r * r)
    return pl.pallas_call(
        kern,
        grid=(n // rb,),
        in_specs=[
            pl.BlockSpec((rb, f), lambda i: (i, 0)),
            pl.BlockSpec((n_pad, f), lambda i: (0, 0)),
            pl.BlockSpec((rb, 1), lambda i: (i, 0)),
            pl.BlockSpec((1, n_pad), lambda i: (0, 0)),
        ],
        out_specs=[
            pl.BlockSpec((rb, K), lambda i: (i, 0)),
            pl.BlockSpec((rb, K), lambda i: (i, 0)),
        ],
        out_shape=[
            jax.ShapeDtypeStruct((n, K), jnp.int32),
            jax.ShapeDtypeStruct((n, K), jnp.float32),
        ],
    )(feat, featp, sq[:, None], sqp[None, :])


_NC, _NS = 2, 16
_NW = _NC * _NS


def _sc_gather_sum(table, idx_flat, n_pad, f, rb=16):
    rpw = n_pad // _NW
    nb = rpw // rb
    assert nb % 2 == 0

    @functools.partial(
        pl.kernel,
        mesh=plsc.VectorSubcoreMesh(core_axis_name="c", subcore_axis_name="s"),
        out_type=jax.ShapeDtypeStruct((n_pad, f), jnp.float32),
        scratch_types=[
            pltpu.VMEM((rpw * K,), jnp.int32),
            pltpu.VMEM((rb * K, f), jnp.float32),
            pltpu.VMEM((rb * K, f), jnp.float32),
            pltpu.VMEM((rb, f), jnp.float32),
            pltpu.SemaphoreType.DMA,
            pltpu.SemaphoreType.DMA,
        ],
    )
    def kern(table_hbm, idx_hbm, out_hbm, idx_v, rows0_v, rows1_v, out_v,
             sem0, sem1):
        wid = lax.axis_index("s") * _NC + lax.axis_index("c")
        base = wid * rpw
        pltpu.sync_copy(idx_hbm.at[pl.ds(base * K, rpw * K)], idx_v)

        def gather(bi, rows_v, sem):
            return pltpu.async_copy(
                table_hbm.at[idx_v.at[pl.ds(bi * rb * K, rb * K)]], rows_v, sem)

        def compute(bi, rows_v):
            for r in range(rb):
                for c in range(f // 16):
                    sl = pl.ds(c * 16, 16)
                    v = rows_v[r * K, sl]
                    for j in range(1, K):
                        v = v + rows_v[r * K + j, sl]
                    out_v[r, sl] = v
            pltpu.sync_copy(out_v, out_hbm.at[pl.ds(base + bi * rb, rb)])

        gather(0, rows0_v, sem0)
        gather(1, rows1_v, sem1)

        def pair(t, carry):
            pltpu.make_async_copy(
                table_hbm.at[idx_v.at[pl.ds(0, rb * K)]], rows0_v, sem0).wait()
            compute(2 * t, rows0_v)

            @pl.when(t < nb // 2 - 1)
            def _():
                gather(2 * t + 2, rows0_v, sem0)

            pltpu.make_async_copy(
                table_hbm.at[idx_v.at[pl.ds(0, rb * K)]], rows1_v, sem1).wait()
            compute(2 * t + 1, rows1_v)

            @pl.when(t < nb // 2 - 1)
            def _():
                gather(2 * t + 3, rows1_v, sem1)

            return carry

        lax.fori_loop(0, nb // 2, pair, 0)

    return kern(table, idx_flat)


def _sage_sc_pair(xc, xt, c_idx, c_valid, t_idx, t_valid, jc, jt, p):
    npc, npt = 10240, 1024
    off = N_CELL + 1
    zc = xc @ p['c%d_wl' % jc].T
    zt = xt @ p['c%d_wl' % jt].T
    f = zc.shape[1]
    table = jnp.concatenate(
        [zc, jnp.zeros((1, f), zc.dtype), zt, jnp.zeros((1, f), zt.dtype)], 0)
    ci = jnp.where(c_valid > 0.0, c_idx, N_CELL).astype(jnp.int32)
    ti = off + jnp.where(t_valid > 0.0, t_idx, N_TISSUE).astype(jnp.int32)
    idxp = jnp.concatenate([
        ci.reshape(-1),
        jnp.full(((npc - N_CELL) * K,), N_CELL, jnp.int32),
        ti.reshape(-1),
        jnp.full(((npt - N_TISSUE) * K,), N_CELL, jnp.int32),
    ])
    s = _sc_gather_sum(table, idxp, npc + npt, f)
    cc = c_valid.sum(1)
    ct = t_valid.sum(1)
    mc = s[:N_CELL] / jnp.maximum(cc, 1.0)[:, None]
    mt = s[npc:npc + N_TISSUE] / jnp.maximum(ct, 1.0)[:, None]
    yc = mc + xc @ p['c%d_wr' % jc].T + p['c%d_b' % jc]
    yt = mt + xt @ p['c%d_wr' % jt].T + p['c%d_b' % jt]
    return yc, yt


_DH = OUT // NHEAD
_SCALE = float(_DH) ** 0.5

_ENC_PNAMES = []
for _l in range(2):
    _ENC_PNAMES += ['t%d_%s' % (_l, s) for s in
                    ('in_w', 'in_b', 'out_w', 'out_b', 'ln1_g', 'ln1_b',
                     'ff1_w', 'ff1_b', 'ff2_w', 'ff2_b', 'ln2_g', 'ln2_b')]
_ENC_PNAMES += ['att1_w', 'att1_b', 'att2_w', 'att2_b']


def _ln_rows(x, g, b):
    m = jnp.mean(x, axis=1, keepdims=True)
    d = x - m
    v = jnp.mean(d * d, axis=1, keepdims=True)
    return d * jax.lax.rsqrt(v + 1e-5) * g + b


def _enc_kern(cls_ref, pos_ref, xc_ref, xt_ref, *refs):
    pr = {n: r[...] for n, r in zip(_ENC_PNAMES, refs[:len(_ENC_PNAMES)])}
    att_ref, pool_ref = refs[len(_ENC_PNAMES):]
    rb = xc_ref.shape[0]
    lane = jax.lax.broadcasted_iota(jnp.int32, (OUT, NHEAD), 0)
    head = jax.lax.broadcasted_iota(jnp.int32, (OUT, NHEAD), 1)
    G = (lane // _DH == head).astype(jnp.float32)

    t = [jnp.broadcast_to(cls_ref[...] + pos_ref[0:1, :], (rb, OUT)),
         xc_ref[...] + pos_ref[1:2, :],
         xt_ref[...] + pos_ref[2:3, :]]

    for l in range(2):
        in_w = pr['t%d_in_w' % l]
        wq, wk, wv = in_w[:OUT], in_w[OUT:2 * OUT], in_w[2 * OUT:]
        in_b = pr['t%d_in_b' % l]
        bq, bk, bv = in_b[:OUT], in_b[OUT:2 * OUT], in_b[2 * OUT:]
        q = [_bdot(ti, wq) + bq for ti in t]
        k = [_bdot(ti, wk) + bk for ti in t]
        v = [_bdot(ti, wv) + bv for ti in t]
        logit = [[jax.lax.dot_general(
            q[i] * k[j], G, (((1,), (0,)), ((), ())),
            preferred_element_type=jnp.float32) / _SCALE
            for j in range(3)] for i in range(3)]
        a = []
        for i in range(3):
            m = jnp.maximum(jnp.maximum(logit[i][0], logit[i][1]), logit[i][2])
            e = [jnp.exp(logit[i][j] - m) for j in range(3)]
            tot = e[0] + e[1] + e[2]
            o = None
            for j in range(3):
                s_full = jax.lax.dot_general(
                    e[j] / tot, G, (((1,), (1,)), ((), ())),
                    preferred_element_type=jnp.float32)
                term = s_full * v[j]
                o = term if o is None else o + term
            a.append(_bdot(o, pr['t%d_out_w' % l]) + pr['t%d_out_b' % l])
        x = [_ln_rows(t[i] + a[i], pr['t%d_ln1_g' % l], pr['t%d_ln1_b' % l])
             for i in range(3)]
        t = []
        for i in range(3):
            f = jnp.maximum(_bdot(x[i], pr['t%d_ff1_w' % l]) + pr['t%d_ff1_b' % l], 0.0)
            f = _bdot(f, pr['t%d_ff2_w' % l]) + pr['t%d_ff2_b' % l]
            t.append(_ln_rows(x[i] + f, pr['t%d_ln2_g' % l], pr['t%d_ln2_b' % l]))

    feat = t[0]
    h = jax.nn.leaky_relu(_bdot(feat, pr['att1_w']) + pr['att1_b'], 0.01)
    att_ref[...] = jnp.sum(h * pr['att2_w'], axis=1, keepdims=True) + pr['att2_b']
    pool_ref[...] = jnp.sum(feat, axis=0, keepdims=True)[None]


def _encoder_head(xc, xtg, p, rb=1000):
    n = xc.shape[0]
    grid = (n // rb,)
    enc_params = [p[name] for name in _ENC_PNAMES]
    cls2 = p['cls'].reshape(1, OUT)
    pos2 = p['pos'].reshape(3, OUT)
    full = lambda a: pl.BlockSpec(a.shape, lambda i: (0,) * a.ndim)
    att_raw, pools = pl.pallas_call(
        _enc_kern,
        grid=grid,
        in_specs=[
            full(cls2),
            full(pos2),
            pl.BlockSpec((rb, OUT), lambda i: (i, 0)),
            pl.BlockSpec((rb, OUT), lambda i: (i, 0)),
        ] + [full(a) for a in enc_params],
        out_specs=[
            pl.BlockSpec((rb, 1), lambda i: (i, 0)),
            pl.BlockSpec((1, 1, OUT), lambda i: (i, 0, 0)),
        ],
        out_shape=[
            jax.ShapeDtypeStruct((n, 1), jnp.float32),
            jax.ShapeDtypeStruct((grid[0], 1, OUT), jnp.float32),
        ],
    )(cls2, pos2, xc, xtg, *enc_params)
    return att_raw, pools.reshape(grid[0], OUT)


def kernel(x_cell, centroids_cell, x_tissue_3, centroids_tissue_3, assignment_matrix_3, params):
    p = params
    batch_idx = jnp.argmax(assignment_matrix_3, axis=1)

    ca = _attr_tower(x_cell, p['ca1_w'], p['ca1_b'], p['ca2_w'], p['ca2_b'])
    cl = _bn(centroids_cell, p['cl_bn_g'], p['cl_bn_b'])
    cl = _lin(_lrelu(_lin(cl, p['cl1_w'], p['cl1_b'])), p['cl2_w'], p['cl2_b'])
    cell_feat = jnp.concatenate([ca, cl], axis=1)
    ta = _attr_tower(x_tissue_3, p['ta1_w'], p['ta1_b'], p['ta2_w'], p['ta2_b'])
    tl = _bn(centroids_tissue_3, p['tl_bn_g'], p['tl_bn_b'])
    tl = _lin(_lrelu(_lin(tl, p['tl1_w'], p['tl1_b'])), p['tl2_w'], p['tl2_b'])
    tis_feat = jnp.concatenate([ta, tl], axis=1)

    c_idx, c_valid = _radius_topk(cell_feat, R, rb=1000, ct=512)
    t_idx, t_valid = _radius_topk(tis_feat, R, rb=1000, ct=512)

    xc, xt = x_cell, x_tissue_3
    for jc, jt in ((1, 4), (2, 5), (3, 6)):
        xc, xt = _sage_sc_pair(xc, xt, c_idx, c_valid, t_idx, t_valid, jc, jt, p)
        xc = _lrelu(_graph_norm(xc, p['gn_g'], p['gn_b'], p['gn_a']))
        xt = _lrelu(_graph_norm(xt, p['gn_g'], p['gn_b'], p['gn_a']))

    xtg = xt[batch_idx]
    att_raw, pools = _encoder_head(xc, xtg, p)
    att = jax.nn.softmax(att_raw, axis=0)
    pooled = jnp.sum(pools, axis=0, keepdims=True) * (1.0 / N_CELL)
    h = _lrelu(_lin(pooled, p['lin1_w'], p['lin1_b']))
    h = _layer_norm(h, p['ln2_g'], p['ln2_b'])
    logits = _lin(h, p['lin2_w'], p['lin2_b'])
    c_src = c_idx.reshape(-1)
    c_dst = jnp.broadcast_to(
        jnp.arange(N_CELL, dtype=c_idx.dtype)[:, None], (N_CELL, K)).reshape(-1)
    edge_index_cell = jnp.stack([c_src, c_dst])
    return logits, edge_index_cell, att
